# baseline (speedup 1.0000x reference)
import functools

import jax
import jax.numpy as jnp
from jax import lax
from jax.experimental import pallas as pl
from jax.experimental.pallas import tpu as pltpu
from jax.experimental.pallas import tpu_sc as plsc

N_NODES = 10000
N_EDGES = 320000
D = 128
NC = 2
NS = 16
DH = D // NC
ROWS_PER_TILE = 640
N_PAD = NS * ROWS_PER_TILE
ROWS_BLK = 128
NBLK = ROWS_PER_TILE // ROWS_BLK
E_PER_TILE = N_EDGES // NS
CHUNK = 128
NCHUNK = 160
E_TILE_PAD = NCHUNK * CHUNK
VPR = DH // 16
NBUF = 4
LOOK = 3
LAG = 1


def _rsqrt16(d):
    y = 1.0 / d
    for _ in range(25):
        y = y * (1.5 - 0.5 * d * y * y)
    return jnp.where(d > 0.5, y, 0.0)


def _sc_body(x_hbm, row_hbm, col_hbm, out_hbm,
             s_hbm, t_sh, deg_sh, row_v, col_v, blk_v, blk16_v,
             gb0, gb1, gb2, gb3, sb0, sb1, dinv_v, ones_v, gsem, ssem, dsem):
    gbufs = (gb0, gb1, gb2, gb3)
    sbufs = (sb0, sb1)
    c = lax.axis_index("c")
    s = lax.axis_index("s")
    base_n = s * ROWS_PER_TILE
    s_base = c * N_PAD

    pltpu.sync_copy(row_hbm.at[s], row_v)
    pltpu.sync_copy(col_hbm.at[s], col_v)

    def _off(j, carry):
        for k in range(CHUNK // 16):
            row_v[j, pl.ds(k * 16, 16)] = row_v[j, pl.ds(k * 16, 16)] + s_base
        return carry
    lax.fori_loop(0, NCHUNK, _off, 0)

    zeros16 = jnp.zeros((16,), jnp.float32)
    for i in range(CHUNK // 16):
        ones_v[pl.ds(i * 16, 16)] = jnp.full((16,), 1.0, jnp.float32)

    def _zero_dinv(i, carry):
        dinv_v[pl.ds(i * 16, 16)] = zeros16
        return carry
    lax.fori_loop(0, ROWS_PER_TILE // 16, _zero_dinv, 0)
    pltpu.sync_copy(dinv_v.at[pl.ds(0, ROWS_PER_TILE)],
                    deg_sh.at[pl.ds(base_n, ROWS_PER_TILE)])

    def _zero_blk(g, carry):
        for u in range(4):
            for k in range(VPR):
                blk_v[g * 4 + u, pl.ds(k * 16, 16)] = zeros16
        return carry

    def _zero_t():
        lax.fori_loop(0, ROWS_BLK // 4, _zero_blk, 0)
        for blk in range(NBLK):
            pltpu.sync_copy(
                blk_v, t_sh.at[pl.ds(base_n + blk * ROWS_BLK, ROWS_BLK), :])
    _zero_t()

    plsc.subcore_barrier()

    def _deg(g, carry):
        for u in range(32):
            pltpu.async_copy(ones_v, deg_sh.at[col_v.at[g * 32 + u]], dsem,
                             add=True)
        for u in range(32):
            pltpu.make_async_copy(ones_v, deg_sh.at[col_v.at[g * 32 + u]],
                                  dsem).wait()
        return carry
    lax.fori_loop(0, NCHUNK // 32, _deg, 0)

    plsc.subcore_barrier()

    pltpu.sync_copy(deg_sh.at[pl.ds(base_n, ROWS_PER_TILE)],
                    dinv_v.at[pl.ds(0, ROWS_PER_TILE)])

    def _newton(i, carry):
        d = dinv_v[pl.ds(i * 16, 16)]
        dinv_v[pl.ds(i * 16, 16)] = _rsqrt16(d)
        return carry
    lax.fori_loop(0, ROWS_PER_TILE // 16, _newton, 0)

    def _scale_blocks(src_ref, src2d, dst_ref, dst2d, squared, row_limit=None,
                      to_bf16=False):
        for blk in range(NBLK):
            b0 = blk * ROWS_BLK
            if row_limit is None:
                off = b0
            else:
                off = jnp.minimum(base_n + b0, row_limit - ROWS_BLK) - base_n

            def _sl(ref, off2d):
                ro, co = off2d
                if co is None:
                    return ref.at[pl.ds(ro + off, ROWS_BLK), :]
                return ref.at[pl.ds(ro + off, ROWS_BLK), pl.ds(co, DH)]

            pltpu.sync_copy(_sl(src_ref, src2d), blk_v)

            def _scale_row(g, carry):
                for u in range(4):
                    r = g * 4 + u
                    dv = dinv_v[pl.ds(off + r, 16)][0]
                    dv = dv * dv if squared else dv
                    for k in range(VPR):
                        blk_v[r, pl.ds(k * 16, 16)] = (
                            blk_v[r, pl.ds(k * 16, 16)] * dv)
                return carry
            lax.fori_loop(0, ROWS_BLK // 4, _scale_row, 0)
            if to_bf16:
                def _pack_row(g, carry):
                    for u in range(4):
                        r = g * 4 + u
                        for k in range(2):
                            pk = plsc.pack(
                                blk_v[r, pl.ds(k * 32, 16)],
                                blk_v[r, pl.ds(k * 32 + 16, 16)],
                                format=plsc.PackFormat.INTERLEAVED)
                            blk16_v[r, pl.ds(k * 32, 32)] = pk
                    return carry
                lax.fori_loop(0, ROWS_BLK // 4, _pack_row, 0)
                pltpu.sync_copy(blk16_v, _sl(dst_ref, dst2d))
            else:
                pltpu.sync_copy(blk_v, _sl(dst_ref, dst2d))

    _scale_blocks(x_hbm, (base_n, c * DH), s_hbm, (s_base + base_n, None),
                  False, row_limit=N_NODES, to_bf16=True)

    plsc.subcore_barrier()

    def _edge_pass():
        for b in range(LOOK):
            pltpu.async_copy(s_hbm.at[row_v.at[b]], gbufs[b], gsem)

        def _group(g, carry):
            for u in range(NBUF):
                j = g * NBUF + u
                buf = gbufs[u]
                sbuf = sbufs[u % 2]
                pltpu.make_async_copy(s_hbm.at[row_v.at[j]], buf, gsem).wait()

                def _conv(g2, carry2):
                    for v in range(4):
                        r = g2 * 4 + v
                        for k in range(2):
                            a, bb = plsc.unpack(
                                buf[r, pl.ds(k * 32, 32)],
                                format=plsc.PackFormat.INTERLEAVED)
                            sbuf[r, pl.ds(k * 32, 16)] = a
                            sbuf[r, pl.ds(k * 32 + 16, 16)] = bb
                    return carry2
                lax.fori_loop(0, CHUNK // 4, _conv, 0)

                pltpu.async_copy(sbuf, t_sh.at[col_v.at[j]], ssem, add=True)

                @pl.when(j >= LAG)
                def _():
                    pltpu.make_async_copy(
                        sbufs[(u + 2 - LAG) % 2],
                        t_sh.at[col_v.at[j - LAG]], ssem).wait()

                @pl.when(j + LOOK < NCHUNK)
                def _():
                    pltpu.async_copy(
                        s_hbm.at[row_v.at[j + LOOK]],
                        gbufs[(u + LOOK) % NBUF], gsem)
            return carry
        lax.fori_loop(0, NCHUNK // NBUF, _group, 0)
        for t in range(LAG):
            pltpu.make_async_copy(
                sbufs[(NCHUNK - LAG + t) % 2],
                t_sh.at[col_v.at[NCHUNK - LAG + t]], ssem).wait()

    _edge_pass()

    plsc.subcore_barrier()

    _scale_blocks(t_sh, (base_n, None), s_hbm, (s_base + base_n, None), True,
                  to_bf16=True)
    _zero_t()

    plsc.subcore_barrier()

    _edge_pass()

    plsc.subcore_barrier()

    _scale_blocks(t_sh, (base_n, None), out_hbm, (base_n, c * DH), False)


_sgc_sc = functools.partial(
    pl.kernel,
    out_type=jax.ShapeDtypeStruct((N_PAD, D), jnp.float32),
    mesh=plsc.VectorSubcoreMesh(core_axis_name="c", subcore_axis_name="s"),
    compiler_params=pltpu.CompilerParams(use_tc_tiling_on_sc=False,
                                         needs_layout_passes=False),
    scratch_types=[
        pltpu.HBM((NC * N_PAD, DH), jnp.bfloat16),
        pltpu.VMEM_SHARED((N_PAD, DH), jnp.float32),
        pltpu.VMEM_SHARED((N_PAD,), jnp.float32),
        pltpu.VMEM((NCHUNK, CHUNK), jnp.int32),
        pltpu.VMEM((NCHUNK, CHUNK), jnp.int32),
        pltpu.VMEM((ROWS_BLK, DH), jnp.float32),
        pltpu.VMEM((ROWS_BLK, DH), jnp.bfloat16),
        pltpu.VMEM((CHUNK, DH), jnp.bfloat16),
        pltpu.VMEM((CHUNK, DH), jnp.bfloat16),
        pltpu.VMEM((CHUNK, DH), jnp.bfloat16),
        pltpu.VMEM((CHUNK, DH), jnp.bfloat16),
        pltpu.VMEM((CHUNK, DH), jnp.float32),
        pltpu.VMEM((CHUNK, DH), jnp.float32),
        pltpu.VMEM((ROWS_PER_TILE + 16,), jnp.float32),
        pltpu.VMEM((CHUNK,), jnp.float32),
        pltpu.SemaphoreType.DMA,
        pltpu.SemaphoreType.DMA,
        pltpu.SemaphoreType.DMA,
    ],
)(_sc_body)


def _mm_body(h_ref, w_ref, b_ref, o_ref):
    o_ref[...] = (
        jnp.dot(h_ref[...], w_ref[...], preferred_element_type=jnp.float32)
        + b_ref[...]
    )


def _linear(h, wt, b2):
    return pl.pallas_call(
        _mm_body,
        grid=(N_NODES // 1000,),
        in_specs=[
            pl.BlockSpec((1000, D), lambda i: (i, 0)),
            pl.BlockSpec((D, D), lambda i: (0, 0)),
            pl.BlockSpec((1, D), lambda i: (0, 0)),
        ],
        out_specs=pl.BlockSpec((1000, D), lambda i: (i, 0)),
        out_shape=jax.ShapeDtypeStruct((N_NODES, D), jnp.float32),
    )(h, wt, b2)


def kernel(x, edge_index, W, b):
    x = x.astype(jnp.float32)
    row = edge_index[0].astype(jnp.int32)
    col = edge_index[1].astype(jnp.int32)

    npad = E_TILE_PAD - E_PER_TILE
    pad_rows = jnp.arange(npad, dtype=jnp.int32) % N_NODES
    pad_cols = N_NODES + (jnp.arange(npad, dtype=jnp.int32)
                          % (N_PAD - N_NODES))
    row_pad = jnp.concatenate(
        [row.reshape(NS, E_PER_TILE),
         jnp.broadcast_to(pad_rows, (NS, npad))], axis=1
    ).reshape(NS, NCHUNK, CHUNK)
    col_pad = jnp.concatenate(
        [col.reshape(NS, E_PER_TILE),
         jnp.broadcast_to(pad_cols, (NS, npad))], axis=1
    ).reshape(NS, NCHUNK, CHUNK)

    h2 = _sgc_sc(x, row_pad, col_pad)
    return _linear(h2, W.T, b.reshape(1, D))


# speedup vs baseline: 1.9151x; 1.9151x over previous
import functools

import jax
import jax.numpy as jnp
from jax import lax
from jax.experimental import pallas as pl
from jax.experimental.pallas import tpu as pltpu
from jax.experimental.pallas import tpu_sc as plsc

N_NODES = 10000
N_EDGES = 320000
D = 128
NC = 2
NS = 16
DH = D // NC
ROWS_PER_TILE = 640
N_PAD = NS * ROWS_PER_TILE
ROWS_BLK = 80
NBLK = ROWS_PER_TILE // ROWS_BLK
E_PER_TILE = N_EDGES // NS
CHUNK = 128
NCHUNK = 160
E_TILE_PAD = NCHUNK * CHUNK
VPR = DH // 16
NBUF = 5
LOOK = 4
LAG = 1


def _rsqrt16(d):
    y = 1.0 / d
    for _ in range(25):
        y = y * (1.5 - 0.5 * d * y * y)
    return jnp.where(d > 0.5, y, 0.0)


def _sc_body(x_hbm, row_hbm, col_hbm, out_hbm,
             s_hbm, t_sh, deg_sh, row_v, col_v, blk_v,
             gb0, gb1, gb2, gb3, gb4, dinv_v, ones_v, gsem, ssem, dsem):
    gbufs = (gb0, gb1, gb2, gb3, gb4)
    c = lax.axis_index("c")
    s = lax.axis_index("s")
    base_n = s * ROWS_PER_TILE
    s_base = c * N_PAD

    pltpu.sync_copy(row_hbm.at[s], row_v)
    pltpu.sync_copy(col_hbm.at[s], col_v)

    def _off(j, carry):
        for k in range(CHUNK // 16):
            row_v[j, pl.ds(k * 16, 16)] = row_v[j, pl.ds(k * 16, 16)] + s_base
        return carry
    lax.fori_loop(0, NCHUNK, _off, 0)

    zeros16 = jnp.zeros((16,), jnp.float32)
    for i in range(CHUNK // 16):
        ones_v[pl.ds(i * 16, 16)] = jnp.full((16,), 1.0, jnp.float32)

    def _zero_dinv(i, carry):
        dinv_v[pl.ds(i * 16, 16)] = zeros16
        return carry
    lax.fori_loop(0, ROWS_PER_TILE // 16, _zero_dinv, 0)
    pltpu.sync_copy(dinv_v.at[pl.ds(0, ROWS_PER_TILE)],
                    deg_sh.at[pl.ds(base_n, ROWS_PER_TILE)])

    def _zero_blk(g, carry):
        for u in range(4):
            for k in range(VPR):
                blk_v[g * 4 + u, pl.ds(k * 16, 16)] = zeros16
        return carry

    def _zero_t():
        lax.fori_loop(0, ROWS_BLK // 4, _zero_blk, 0)
        for blk in range(NBLK):
            pltpu.sync_copy(
                blk_v, t_sh.at[pl.ds(base_n + blk * ROWS_BLK, ROWS_BLK), :])
    _zero_t()

    plsc.subcore_barrier()

    def _deg(g, carry):
        for u in range(32):
            pltpu.async_copy(ones_v, deg_sh.at[col_v.at[g * 32 + u]], dsem,
                             add=True)
        for u in range(32):
            pltpu.make_async_copy(ones_v, deg_sh.at[col_v.at[g * 32 + u]],
                                  dsem).wait()
        return carry
    lax.fori_loop(0, NCHUNK // 32, _deg, 0)

    plsc.subcore_barrier()

    pltpu.sync_copy(deg_sh.at[pl.ds(base_n, ROWS_PER_TILE)],
                    dinv_v.at[pl.ds(0, ROWS_PER_TILE)])

    def _newton(i, carry):
        d = dinv_v[pl.ds(i * 16, 16)]
        dinv_v[pl.ds(i * 16, 16)] = _rsqrt16(d)
        return carry
    lax.fori_loop(0, ROWS_PER_TILE // 16, _newton, 0)

    def _scale_blocks(src_ref, src2d, dst_ref, dst2d, squared, row_limit=None):
        for blk in range(NBLK):
            b0 = blk * ROWS_BLK
            if row_limit is None:
                off = b0
            else:
                off = jnp.minimum(base_n + b0, row_limit - ROWS_BLK) - base_n

            def _sl(ref, off2d):
                ro, co = off2d
                if co is None:
                    return ref.at[pl.ds(ro + off, ROWS_BLK), :]
                return ref.at[pl.ds(ro + off, ROWS_BLK), pl.ds(co, DH)]

            pltpu.sync_copy(_sl(src_ref, src2d), blk_v)

            def _scale_row(g, carry):
                for u in range(4):
                    r = g * 4 + u
                    dv = dinv_v[pl.ds(off + r, 16)][0]
                    dv = dv * dv if squared else dv
                    for k in range(VPR):
                        blk_v[r, pl.ds(k * 16, 16)] = (
                            blk_v[r, pl.ds(k * 16, 16)] * dv)
                return carry
            lax.fori_loop(0, ROWS_BLK // 4, _scale_row, 0)
            pltpu.sync_copy(blk_v, _sl(dst_ref, dst2d))

    _scale_blocks(x_hbm, (base_n, c * DH), s_hbm, (s_base + base_n, None),
                  False, row_limit=N_NODES)

    plsc.subcore_barrier()

    def _edge_pass():
        for b in range(LOOK):
            pltpu.async_copy(s_hbm.at[row_v.at[b]], gbufs[b], gsem)

        def _group(g, carry):
            for u in range(NBUF):
                j = g * NBUF + u
                buf = gbufs[u]
                pltpu.make_async_copy(s_hbm.at[row_v.at[j]], buf, gsem).wait()
                pltpu.async_copy(buf, t_sh.at[col_v.at[j]], ssem, add=True)

                @pl.when(j >= LAG)
                def _():
                    pltpu.make_async_copy(
                        gbufs[(u + NBUF - LAG) % NBUF],
                        t_sh.at[col_v.at[j - LAG]], ssem).wait()

                @pl.when(j + LOOK < NCHUNK)
                def _():
                    pltpu.async_copy(
                        s_hbm.at[row_v.at[j + LOOK]],
                        gbufs[(u + LOOK) % NBUF], gsem)
            return carry
        lax.fori_loop(0, NCHUNK // NBUF, _group, 0)
        for t in range(LAG):
            pltpu.make_async_copy(
                gbufs[(NCHUNK - LAG + t) % NBUF],
                t_sh.at[col_v.at[NCHUNK - LAG + t]], ssem).wait()

    _edge_pass()

    plsc.subcore_barrier()

    _scale_blocks(t_sh, (base_n, None), s_hbm, (s_base + base_n, None), True)
    _zero_t()

    plsc.subcore_barrier()

    _edge_pass()

    plsc.subcore_barrier()

    _scale_blocks(t_sh, (base_n, None), out_hbm, (base_n, c * DH), False)


_sgc_sc = functools.partial(
    pl.kernel,
    out_type=jax.ShapeDtypeStruct((N_PAD, D), jnp.float32),
    mesh=plsc.VectorSubcoreMesh(core_axis_name="c", subcore_axis_name="s"),
    compiler_params=pltpu.CompilerParams(use_tc_tiling_on_sc=False),
    scratch_types=[
        pltpu.HBM((NC * N_PAD, DH), jnp.float32),
        pltpu.VMEM_SHARED((N_PAD, DH), jnp.float32),
        pltpu.VMEM_SHARED((N_PAD,), jnp.float32),
        pltpu.VMEM((NCHUNK, CHUNK), jnp.int32),
        pltpu.VMEM((NCHUNK, CHUNK), jnp.int32),
        pltpu.VMEM((ROWS_BLK, DH), jnp.float32),
        pltpu.VMEM((CHUNK, DH), jnp.float32),
        pltpu.VMEM((CHUNK, DH), jnp.float32),
        pltpu.VMEM((CHUNK, DH), jnp.float32),
        pltpu.VMEM((CHUNK, DH), jnp.float32),
        pltpu.VMEM((CHUNK, DH), jnp.float32),
        pltpu.VMEM((ROWS_PER_TILE + 16,), jnp.float32),
        pltpu.VMEM((CHUNK,), jnp.float32),
        pltpu.SemaphoreType.DMA,
        pltpu.SemaphoreType.DMA,
        pltpu.SemaphoreType.DMA,
    ],
)(_sc_body)


def _mm_body(h_ref, w_ref, b_ref, o_ref):
    o_ref[...] = (
        jnp.dot(h_ref[...], w_ref[...], preferred_element_type=jnp.float32)
        + b_ref[...]
    )


def _linear(h, wt, b2):
    return pl.pallas_call(
        _mm_body,
        grid=(N_NODES // 1000,),
        in_specs=[
            pl.BlockSpec((1000, D), lambda i: (i, 0)),
            pl.BlockSpec((D, D), lambda i: (0, 0)),
            pl.BlockSpec((1, D), lambda i: (0, 0)),
        ],
        out_specs=pl.BlockSpec((1000, D), lambda i: (i, 0)),
        out_shape=jax.ShapeDtypeStruct((N_NODES, D), jnp.float32),
    )(h, wt, b2)


def kernel(x, edge_index, W, b):
    x = x.astype(jnp.float32)
    row = edge_index[0].astype(jnp.int32)
    col = edge_index[1].astype(jnp.int32)

    npad = E_TILE_PAD - E_PER_TILE
    pad_rows = jnp.arange(npad, dtype=jnp.int32) % N_NODES
    pad_cols = N_NODES + (jnp.arange(npad, dtype=jnp.int32)
                          % (N_PAD - N_NODES))
    row_pad = jnp.concatenate(
        [row.reshape(NS, E_PER_TILE),
         jnp.broadcast_to(pad_rows, (NS, npad))], axis=1
    ).reshape(NS, NCHUNK, CHUNK)
    col_pad = jnp.concatenate(
        [col.reshape(NS, E_PER_TILE),
         jnp.broadcast_to(pad_cols, (NS, npad))], axis=1
    ).reshape(NS, NCHUNK, CHUNK)

    h2 = _sgc_sc(x, row_pad, col_pad)
    return _linear(h2, W.T, b.reshape(1, D))


# speedup vs baseline: 1.9438x; 1.0150x over previous
import functools

import jax
import jax.numpy as jnp
from jax import lax
from jax.experimental import pallas as pl
from jax.experimental.pallas import tpu as pltpu
from jax.experimental.pallas import tpu_sc as plsc

N_NODES = 10000
N_EDGES = 320000
D = 128
NC = 2
NS = 16
DH = D // NC
ROWS_PER_TILE = 640
N_PAD = NS * ROWS_PER_TILE
ROWS_BLK = 128
NBLK = ROWS_PER_TILE // ROWS_BLK
E_PER_TILE = N_EDGES // NS
CHUNK = 128
NCHUNK = 160
E_TILE_PAD = NCHUNK * CHUNK
VPR = DH // 16
NBUF = 4
LOOK = 3
LAG = 1


def _rsqrt16(d):
    y = 1.0 / d
    for _ in range(25):
        y = y * (1.5 - 0.5 * d * y * y)
    return jnp.where(d > 0.5, y, 0.0)


def _sc_body(x_hbm, row_hbm, col_hbm, out_hbm,
             s_hbm, t_sh, deg_sh, row_v, col_v, blk_v,
             gb0, gb1, gb2, gb3, dinv_v, ones_v, gsem, ssem, dsem):
    gbufs = (gb0, gb1, gb2, gb3)
    c = lax.axis_index("c")
    s = lax.axis_index("s")
    base_n = s * ROWS_PER_TILE
    s_base = c * N_PAD

    pltpu.sync_copy(row_hbm.at[s], row_v)
    pltpu.sync_copy(col_hbm.at[s], col_v)

    def _off(j, carry):
        for k in range(CHUNK // 16):
            row_v[j, pl.ds(k * 16, 16)] = row_v[j, pl.ds(k * 16, 16)] + s_base
        return carry
    lax.fori_loop(0, NCHUNK, _off, 0)

    zeros16 = jnp.zeros((16,), jnp.float32)
    for i in range(CHUNK // 16):
        ones_v[pl.ds(i * 16, 16)] = jnp.full((16,), 1.0, jnp.float32)

    def _zero_dinv(i, carry):
        dinv_v[pl.ds(i * 16, 16)] = zeros16
        return carry
    lax.fori_loop(0, ROWS_PER_TILE // 16, _zero_dinv, 0)
    pltpu.sync_copy(dinv_v.at[pl.ds(0, ROWS_PER_TILE)],
                    deg_sh.at[pl.ds(base_n, ROWS_PER_TILE)])

    def _zero_blk(g, carry):
        for u in range(4):
            for k in range(VPR):
                blk_v[g * 4 + u, pl.ds(k * 16, 16)] = zeros16
        return carry

    def _zero_t():
        lax.fori_loop(0, ROWS_BLK // 4, _zero_blk, 0)
        for blk in range(NBLK):
            pltpu.sync_copy(
                blk_v, t_sh.at[pl.ds(base_n + blk * ROWS_BLK, ROWS_BLK), :])
    _zero_t()

    plsc.subcore_barrier()

    def _deg(g, carry):
        for u in range(32):
            pltpu.async_copy(ones_v, deg_sh.at[col_v.at[g * 32 + u]], dsem,
                             add=True)
        for u in range(32):
            pltpu.make_async_copy(ones_v, deg_sh.at[col_v.at[g * 32 + u]],
                                  dsem).wait()
        return carry
    lax.fori_loop(0, NCHUNK // 32, _deg, 0)

    plsc.subcore_barrier()

    pltpu.sync_copy(deg_sh.at[pl.ds(base_n, ROWS_PER_TILE)],
                    dinv_v.at[pl.ds(0, ROWS_PER_TILE)])

    def _newton(i, carry):
        d = dinv_v[pl.ds(i * 16, 16)]
        dinv_v[pl.ds(i * 16, 16)] = _rsqrt16(d)
        return carry
    lax.fori_loop(0, ROWS_PER_TILE // 16, _newton, 0)

    def _scale_blocks(src_ref, src2d, dst_ref, dst2d, squared, row_limit=None):
        for blk in range(NBLK):
            b0 = blk * ROWS_BLK
            if row_limit is None:
                off = b0
            else:
                off = jnp.minimum(base_n + b0, row_limit - ROWS_BLK) - base_n

            def _sl(ref, off2d):
                ro, co = off2d
                if co is None:
                    return ref.at[pl.ds(ro + off, ROWS_BLK), :]
                return ref.at[pl.ds(ro + off, ROWS_BLK), pl.ds(co, DH)]

            pltpu.sync_copy(_sl(src_ref, src2d), blk_v)

            def _scale_row(g, carry):
                for u in range(4):
                    r = g * 4 + u
                    dv = dinv_v[pl.ds(off + r, 16)][0]
                    dv = dv * dv if squared else dv
                    for k in range(VPR):
                        blk_v[r, pl.ds(k * 16, 16)] = (
                            blk_v[r, pl.ds(k * 16, 16)] * dv)
                return carry
            lax.fori_loop(0, ROWS_BLK // 4, _scale_row, 0)
            pltpu.sync_copy(blk_v, _sl(dst_ref, dst2d))

    _scale_blocks(x_hbm, (base_n, c * DH), s_hbm, (s_base + base_n, None),
                  False, row_limit=N_NODES)

    plsc.subcore_barrier()

    def _edge_pass():
        for b in range(LOOK):
            pltpu.async_copy(s_hbm.at[row_v.at[b]], gbufs[b], gsem)

        def _group(g, carry):
            for u in range(NBUF):
                j = g * NBUF + u
                buf = gbufs[u]
                pltpu.make_async_copy(s_hbm.at[row_v.at[j]], buf, gsem).wait()
                pltpu.async_copy(buf, t_sh.at[col_v.at[j]], ssem, add=True)

                @pl.when(j >= LAG)
                def _():
                    pltpu.make_async_copy(
                        gbufs[(u + NBUF - LAG) % NBUF],
                        t_sh.at[col_v.at[j - LAG]], ssem).wait()

                @pl.when(j + LOOK < NCHUNK)
                def _():
                    pltpu.async_copy(
                        s_hbm.at[row_v.at[j + LOOK]],
                        gbufs[(u + LOOK) % NBUF], gsem)
            return carry
        lax.fori_loop(0, NCHUNK // NBUF, _group, 0)
        for t in range(LAG):
            pltpu.make_async_copy(
                gbufs[(NCHUNK - LAG + t) % NBUF],
                t_sh.at[col_v.at[NCHUNK - LAG + t]], ssem).wait()

    _edge_pass()

    plsc.subcore_barrier()

    _scale_blocks(t_sh, (base_n, None), s_hbm, (s_base + base_n, None), True)
    _zero_t()

    plsc.subcore_barrier()

    _edge_pass()

    plsc.subcore_barrier()

    _scale_blocks(t_sh, (base_n, None), out_hbm, (base_n, c * DH), False)


_sgc_sc = functools.partial(
    pl.kernel,
    out_type=jax.ShapeDtypeStruct((N_PAD, D), jnp.float32),
    mesh=plsc.VectorSubcoreMesh(core_axis_name="c", subcore_axis_name="s"),
    compiler_params=pltpu.CompilerParams(use_tc_tiling_on_sc=False),
    scratch_types=[
        pltpu.HBM((NC * N_PAD, DH), jnp.float32),
        pltpu.VMEM_SHARED((N_PAD, DH), jnp.float32),
        pltpu.VMEM_SHARED((N_PAD,), jnp.float32),
        pltpu.VMEM((NCHUNK, CHUNK), jnp.int32),
        pltpu.VMEM((NCHUNK, CHUNK), jnp.int32),
        pltpu.VMEM((ROWS_BLK, DH), jnp.float32),
        pltpu.VMEM((CHUNK, DH), jnp.float32),
        pltpu.VMEM((CHUNK, DH), jnp.float32),
        pltpu.VMEM((CHUNK, DH), jnp.float32),
        pltpu.VMEM((CHUNK, DH), jnp.float32),
        pltpu.VMEM((ROWS_PER_TILE + 16,), jnp.float32),
        pltpu.VMEM((CHUNK,), jnp.float32),
        pltpu.SemaphoreType.DMA,
        pltpu.SemaphoreType.DMA,
        pltpu.SemaphoreType.DMA,
    ],
)(_sc_body)


def _mm_body(h_ref, w_ref, b_ref, o_ref):
    o_ref[...] = (
        jnp.dot(h_ref[...], w_ref[...], preferred_element_type=jnp.float32)
        + b_ref[...]
    )


def _linear(h, wt, b2):
    return pl.pallas_call(
        _mm_body,
        grid=(N_NODES // 1000,),
        in_specs=[
            pl.BlockSpec((1000, D), lambda i: (i, 0)),
            pl.BlockSpec((D, D), lambda i: (0, 0)),
            pl.BlockSpec((1, D), lambda i: (0, 0)),
        ],
        out_specs=pl.BlockSpec((1000, D), lambda i: (i, 0)),
        out_shape=jax.ShapeDtypeStruct((N_NODES, D), jnp.float32),
    )(h, wt, b2)


def kernel(x, edge_index, W, b):
    x = x.astype(jnp.float32)
    row = edge_index[0].astype(jnp.int32)
    col = edge_index[1].astype(jnp.int32)

    npad = E_TILE_PAD - E_PER_TILE
    pad_rows = jnp.arange(npad, dtype=jnp.int32) % N_NODES
    pad_cols = N_NODES + (jnp.arange(npad, dtype=jnp.int32)
                          % (N_PAD - N_NODES))
    row_pad = jnp.concatenate(
        [row.reshape(NS, E_PER_TILE),
         jnp.broadcast_to(pad_rows, (NS, npad))], axis=1
    ).reshape(NS, NCHUNK, CHUNK)
    col_pad = jnp.concatenate(
        [col.reshape(NS, E_PER_TILE),
         jnp.broadcast_to(pad_cols, (NS, npad))], axis=1
    ).reshape(NS, NCHUNK, CHUNK)

    h2 = _sgc_sc(x, row_pad, col_pad)
    return _linear(h2, W.T, b.reshape(1, D))
